# baseline (speedup 1.0000x reference)
import jax
import jax.numpy as jnp
from jax import lax
from jax.experimental import pallas as pl
from jax.experimental.pallas import tpu as pltpu
from jax.experimental.pallas import tpu_sc as plsc

N = 4096
M = 8192
H = 256
BN = 512
NB = N // BN
CM = 2048
NC = M // CM

SC_NS = 16
APW = N // SC_NS
CW = M // SC_NS


def _bdot(a, b):
    return jnp.dot(a.astype(jnp.bfloat16), b.astype(jnp.bfloat16),
                   preferred_element_type=jnp.float32)


def _enc_argmin_kernel(posP_ref, gridP_ref, W1_ref, b1_ref, W2_ref, b2_ref,
                       W3_ref, b3_ref, feats_ref, idx_ref, gB_ref, gsq_ref):
    b = pl.program_id(0)

    @pl.when(b == 0)
    def _build_grid_aug():
        g = gridP_ref[...]
        gB_ref[...] = g.astype(jnp.bfloat16)
        gsq_ref[...] = ((g[0:1] * g[0:1] + g[1:2] * g[1:2])
                        + g[2:3] * g[2:3])

    p = posP_ref[...]
    h = jnp.maximum(_bdot(p, W1_ref[...]) + b1_ref[...], 0.0)
    h = jnp.maximum(_bdot(h, W2_ref[...]) + b2_ref[...], 0.0)
    feats_ref[...] = _bdot(h, W3_ref[...]) + b3_ref[...]

    asq = ((p[:, 0:1] * p[:, 0:1] + p[:, 1:2] * p[:, 1:2])
           + p[:, 2:3] * p[:, 2:3])
    pm2_bf = (p * -2.0).astype(jnp.bfloat16)
    gi_f = jax.lax.broadcasted_iota(jnp.int32, (BN, CM), 1).astype(jnp.float32)
    run_min = jnp.full((BN, 1), jnp.inf, jnp.float32)
    run_arg = jnp.zeros((BN, 1), jnp.float32)
    for c in range(NC):
        sl = slice(c * CM, (c + 1) * CM)
        ag2 = jnp.dot(pm2_bf, gB_ref[:, sl],
                      preferred_element_type=jnp.float32)
        d2 = (asq + ag2) + gsq_ref[:, sl]
        mn = jnp.min(d2, axis=1, keepdims=True)
        am = jnp.min(jnp.where(d2 == mn, gi_f, 3e9), axis=1,
                     keepdims=True) + (c * CM)
        upd = mn < run_min
        run_arg = jnp.where(upd, am, run_arg)
        run_min = jnp.minimum(run_min, mn)
    idx_ref[...] = jnp.swapaxes(run_arg.astype(jnp.int32), 0, 1)[None]


def _winner_mask_sc(idx_hbm, w_hbm, idx_v, slot_v, tbuf_v, win_v, wfull_v,
                    wout_v, shared_slots, winner_sh):
    wid = lax.axis_index("s")
    base_a = wid * APW
    pltpu.sync_copy(idx_hbm.at[pl.ds(base_a, APW)], idx_v)
    lane = lax.iota(jnp.int32, 16)

    def _init_body(i, carry):
        slot_v[pl.ds(i * 16, 16)] = jnp.full((16,), -1, jnp.int32)
        return carry
    lax.fori_loop(0, M // 16, _init_body, 0)

    for g in range(APW // 16):
        iv = idx_v[pl.ds(g * 16, 16)]
        val = base_a + g * 16 + lane
        for kk in range(16):
            plsc.store_scatter(slot_v, [iv], val, mask=(lane == kk))

    pltpu.sync_copy(slot_v, shared_slots.at[wid])
    plsc.subcore_barrier()

    base_c = wid * CW
    pltpu.sync_copy(shared_slots.at[0, pl.ds(base_c, CW)], win_v)
    for t in range(1, SC_NS):
        pltpu.sync_copy(shared_slots.at[t, pl.ds(base_c, CW)], tbuf_v)
        for cc in range(CW // 16):
            sl2 = pl.ds(cc * 16, 16)
            win_v[sl2] = jnp.maximum(win_v[sl2], tbuf_v[sl2])
    pltpu.sync_copy(win_v, winner_sh.at[pl.ds(base_c, CW)])
    plsc.subcore_barrier()

    pltpu.sync_copy(winner_sh, wfull_v)
    for g in range(APW // 16):
        iv = idx_v[pl.ds(g * 16, 16)]
        wn = plsc.load_gather(wfull_v, [iv])
        myid = base_a + g * 16 + lane
        wout_v[pl.ds(g * 16, 16)] = jnp.where(wn == myid, 1.0, 0.0)
    pltpu.sync_copy(wout_v, w_hbm.at[pl.ds(base_a, APW)])


def _reduce_kernel(feats_ref, w_ref, Wn1_ref, bn1_ref, Wn2_ref,
                   bn2_ref, Wh1_ref, bh1_ref, Wh2_ref, bh2_ref, out_ref,
                   acc_ref, cnt_ref):
    b = pl.program_id(0)

    @pl.when(b == 0)
    def _init():
        acc_ref[...] = jnp.zeros_like(acc_ref)
        cnt_ref[...] = jnp.zeros_like(cnt_ref)

    w = jnp.swapaxes(w_ref[0], 0, 1)
    z = jnp.maximum(_bdot(feats_ref[...], Wn1_ref[...]) + bn1_ref[...], 0.0)
    acc_ref[...] += jnp.sum(z * w, axis=0, keepdims=True)
    cnt_ref[...] += jnp.sum(w, axis=0, keepdims=True)

    @pl.when(b == NB - 1)
    def _final():
        nocc = cnt_ref[...]
        srel = acc_ref[...] + (M - nocc) * jnp.maximum(bn1_ref[...], 0.0)
        agg = _bdot(srel * (1.0 / M), Wn2_ref[...]) + bn2_ref[...]
        hh = jnp.maximum(_bdot(agg, Wh1_ref[...]) + bh1_ref[...], 0.0)
        prod = (hh.astype(jnp.bfloat16).astype(jnp.float32)
                * Wh2_ref[...].astype(jnp.bfloat16).astype(jnp.float32))
        hw = H // 2
        for k in range(4):
            s = jnp.sum(prod[:, k * hw:(k + 1) * hw], axis=1, keepdims=True)
            out_ref[:, k:k + 1] = s + bh2_ref[:, k:k + 1]


def kernel(atomic_positions, grid_points, W1, b1, W2, b2, W3, b3,
           Wn1, bn1, Wn2, bn2, Wh1, bh1, Wh2, bh2):
    f32 = jnp.float32
    posP = jnp.pad(atomic_positions, ((0, 0), (0, 5)))
    gridP = jnp.pad(grid_points.T, ((0, 5), (0, 0)))
    W1P = jnp.pad(W1, ((0, 5), (0, 0)))
    b1r = b1.reshape(1, -1)
    b2r = b2.reshape(1, -1)
    b3r = b3.reshape(1, -1)
    bn1r = bn1.reshape(1, -1)
    bn2r = bn2.reshape(1, -1)
    Wh1r = Wh1.transpose(1, 0, 2).reshape(H, 4 * (H // 2))
    bh1r = bh1.reshape(1, -1)
    Wh2r = Wh2[:, :, 0].reshape(1, -1)
    bh2r = bh2.reshape(1, -1)

    feats, idx = pl.pallas_call(
        _enc_argmin_kernel,
        grid=(NB,),
        in_specs=[
            pl.BlockSpec((BN, 8), lambda b: (b, 0)),
            pl.BlockSpec((8, M), lambda b: (0, 0)),
            pl.BlockSpec((8, H // 4), lambda b: (0, 0)),
            pl.BlockSpec((1, H // 4), lambda b: (0, 0)),
            pl.BlockSpec((H // 4, H // 2), lambda b: (0, 0)),
            pl.BlockSpec((1, H // 2), lambda b: (0, 0)),
            pl.BlockSpec((H // 2, H), lambda b: (0, 0)),
            pl.BlockSpec((1, H), lambda b: (0, 0)),
        ],
        out_specs=[
            pl.BlockSpec((BN, H), lambda b: (b, 0)),
            pl.BlockSpec((1, 1, BN), lambda b: (b, 0, 0)),
        ],
        out_shape=[
            jax.ShapeDtypeStruct((N, H), f32),
            jax.ShapeDtypeStruct((NB, 1, BN), jnp.int32),
        ],
        scratch_shapes=[pltpu.VMEM((8, M), jnp.bfloat16),
                        pltpu.VMEM((1, M), f32)],
    )(posP, gridP, W1P, b1r, W2, b2r, W3, b3r)

    wmask = pl.kernel(
        _winner_mask_sc,
        out_type=jax.ShapeDtypeStruct((N,), f32),
        mesh=plsc.VectorSubcoreMesh(core_axis_name="c", subcore_axis_name="s",
                                    num_cores=1),
        compiler_params=pltpu.CompilerParams(needs_layout_passes=False),
        scratch_types=[
            pltpu.VMEM((APW,), jnp.int32),
            pltpu.VMEM((M,), jnp.int32),
            pltpu.VMEM((CW,), jnp.int32),
            pltpu.VMEM((CW,), jnp.int32),
            pltpu.VMEM((M,), jnp.int32),
            pltpu.VMEM((APW,), f32),
            pltpu.VMEM_SHARED((SC_NS, M), jnp.int32),
            pltpu.VMEM_SHARED((M,), jnp.int32),
        ],
    )(idx.reshape(N))

    out = pl.pallas_call(
        _reduce_kernel,
        grid=(NB,),
        in_specs=[
            pl.BlockSpec((BN, H), lambda b: (b, 0)),
            pl.BlockSpec((1, 1, BN), lambda b: (b, 0, 0)),
            pl.BlockSpec((H, H), lambda b: (0, 0)),
            pl.BlockSpec((1, H), lambda b: (0, 0)),
            pl.BlockSpec((H, H), lambda b: (0, 0)),
            pl.BlockSpec((1, H), lambda b: (0, 0)),
            pl.BlockSpec((H, 4 * (H // 2)), lambda b: (0, 0)),
            pl.BlockSpec((1, 4 * (H // 2)), lambda b: (0, 0)),
            pl.BlockSpec((1, 4 * (H // 2)), lambda b: (0, 0)),
            pl.BlockSpec((1, 4), lambda b: (0, 0)),
        ],
        out_specs=pl.BlockSpec((1, 4), lambda b: (0, 0)),
        out_shape=jax.ShapeDtypeStruct((1, 4), f32),
        scratch_shapes=[pltpu.VMEM((1, H), f32), pltpu.VMEM((1, 1), f32)],
    )(feats, wmask.reshape(NB, 1, BN), Wn1, bn1r, Wn2, bn2r,
      Wh1r, bh1r, Wh2r, bh2r)

    return out.reshape(4)


# speedup vs baseline: 1.0460x; 1.0460x over previous
import jax
import jax.numpy as jnp
from jax import lax
from jax.experimental import pallas as pl
from jax.experimental.pallas import tpu as pltpu
from jax.experimental.pallas import tpu_sc as plsc

N = 4096
M = 8192
H = 256
BN = 512
NB = N // BN
CM = 2048
NC = M // CM

SC_NS = 16
APW = N // SC_NS
CW = M // SC_NS


def _bdot(a, b):
    return jnp.dot(a.astype(jnp.bfloat16), b.astype(jnp.bfloat16),
                   preferred_element_type=jnp.float32)


def _enc_argmin_kernel(posP_ref, gridP_ref, W1_ref, b1_ref, W2_ref, b2_ref,
                       W3_ref, b3_ref, feats_ref, idx_ref, gB_ref, gsq_ref):
    b = pl.program_id(0)

    @pl.when(b == 0)
    def _build_grid_aug():
        g = gridP_ref[...]
        gB_ref[...] = g.astype(jnp.bfloat16)
        gsq_ref[...] = ((g[0:1] * g[0:1] + g[1:2] * g[1:2])
                        + g[2:3] * g[2:3])

    p = posP_ref[...]
    h = jnp.maximum(_bdot(p, W1_ref[...]) + b1_ref[...], 0.0)
    h = jnp.maximum(_bdot(h, W2_ref[...]) + b2_ref[...], 0.0)
    feats_ref[...] = _bdot(h, W3_ref[...]) + b3_ref[...]

    asq = ((p[:, 0:1] * p[:, 0:1] + p[:, 1:2] * p[:, 1:2])
           + p[:, 2:3] * p[:, 2:3])
    pm2_bf = (p * -2.0).astype(jnp.bfloat16)
    gi_f = jax.lax.broadcasted_iota(jnp.int32, (BN, CM), 1).astype(jnp.float32)
    run_min = jnp.full((BN, 1), jnp.inf, jnp.float32)
    run_arg = jnp.zeros((BN, 1), jnp.float32)
    for c in range(NC):
        sl = slice(c * CM, (c + 1) * CM)
        ag2 = jnp.dot(pm2_bf, gB_ref[:, sl],
                      preferred_element_type=jnp.float32)
        d2 = (asq + ag2) + gsq_ref[:, sl]
        mn = jnp.min(d2, axis=1, keepdims=True)
        am = jnp.min(jnp.where(d2 == mn, gi_f, 3e9), axis=1,
                     keepdims=True) + (c * CM)
        upd = mn < run_min
        run_arg = jnp.where(upd, am, run_arg)
        run_min = jnp.minimum(run_min, mn)
    idx_ref[...] = jnp.swapaxes(run_arg.astype(jnp.int32), 0, 1)[None]


def _winner_mask_sc(idx_hbm, w_hbm, idx_v, slot_v, tbuf_v, win_v, wfull_v,
                    wout_v, shared_slots, winner_sh, sem):
    wid = lax.axis_index("s")
    base_a = wid * APW
    pltpu.sync_copy(idx_hbm.at[pl.ds(base_a, APW)], idx_v)
    lane = lax.iota(jnp.int32, 16)

    neg1 = jnp.full((16,), -1, jnp.int32)
    for i in range(M // 16):
        slot_v[i * 16:(i + 1) * 16] = neg1

    for g in range(APW // 16):
        iv = idx_v[pl.ds(g * 16, 16)]
        val = base_a + g * 16 + lane
        for kk in range(16):
            plsc.store_scatter(slot_v, [iv], val, mask=(lane == kk))

    pltpu.sync_copy(slot_v, shared_slots.at[wid])
    plsc.subcore_barrier()

    base_c = wid * CW
    cps = [pltpu.async_copy(shared_slots.at[t, pl.ds(base_c, CW)],
                            tbuf_v.at[t], sem) for t in range(SC_NS)]
    for cp in cps:
        cp.wait()
    for cc in range(CW // 16):
        sl2 = pl.ds(cc * 16, 16)
        acc16 = tbuf_v[0, cc * 16:(cc + 1) * 16]
        for t in range(1, SC_NS):
            acc16 = jnp.maximum(acc16, tbuf_v[t, cc * 16:(cc + 1) * 16])
        win_v[sl2] = acc16
    pltpu.sync_copy(win_v, winner_sh.at[pl.ds(base_c, CW)])
    plsc.subcore_barrier()

    pltpu.sync_copy(winner_sh, wfull_v)
    for g in range(APW // 16):
        iv = idx_v[pl.ds(g * 16, 16)]
        wn = plsc.load_gather(wfull_v, [iv])
        myid = base_a + g * 16 + lane
        wout_v[pl.ds(g * 16, 16)] = jnp.where(wn == myid, 1.0, 0.0)
    pltpu.sync_copy(wout_v, w_hbm.at[pl.ds(base_a, APW)])


def _reduce_kernel(feats_ref, w_ref, Wn1_ref, bn1_ref, Wn2_ref,
                   bn2_ref, Wh1_ref, bh1_ref, Wh2_ref, bh2_ref, out_ref,
                   acc_ref, cnt_ref):
    b = pl.program_id(0)

    @pl.when(b == 0)
    def _init():
        acc_ref[...] = jnp.zeros_like(acc_ref)
        cnt_ref[...] = jnp.zeros_like(cnt_ref)

    w = jnp.swapaxes(w_ref[0], 0, 1)
    z = jnp.maximum(_bdot(feats_ref[...], Wn1_ref[...]) + bn1_ref[...], 0.0)
    acc_ref[...] += jnp.sum(z * w, axis=0, keepdims=True)
    cnt_ref[...] += jnp.sum(w, axis=0, keepdims=True)

    @pl.when(b == NB - 1)
    def _final():
        nocc = cnt_ref[...]
        srel = acc_ref[...] + (M - nocc) * jnp.maximum(bn1_ref[...], 0.0)
        agg = _bdot(srel * (1.0 / M), Wn2_ref[...]) + bn2_ref[...]
        hh = jnp.maximum(_bdot(agg, Wh1_ref[...]) + bh1_ref[...], 0.0)
        prod = (hh.astype(jnp.bfloat16).astype(jnp.float32)
                * Wh2_ref[...].astype(jnp.bfloat16).astype(jnp.float32))
        hw = H // 2
        for k in range(4):
            s = jnp.sum(prod[:, k * hw:(k + 1) * hw], axis=1, keepdims=True)
            out_ref[:, k:k + 1] = s + bh2_ref[:, k:k + 1]


def kernel(atomic_positions, grid_points, W1, b1, W2, b2, W3, b3,
           Wn1, bn1, Wn2, bn2, Wh1, bh1, Wh2, bh2):
    f32 = jnp.float32
    posP = jnp.pad(atomic_positions, ((0, 0), (0, 5)))
    gridP = jnp.pad(grid_points.T, ((0, 5), (0, 0)))
    W1P = jnp.pad(W1, ((0, 5), (0, 0)))
    b1r = b1.reshape(1, -1)
    b2r = b2.reshape(1, -1)
    b3r = b3.reshape(1, -1)
    bn1r = bn1.reshape(1, -1)
    bn2r = bn2.reshape(1, -1)
    Wh1r = Wh1.transpose(1, 0, 2).reshape(H, 4 * (H // 2))
    bh1r = bh1.reshape(1, -1)
    Wh2r = Wh2[:, :, 0].reshape(1, -1)
    bh2r = bh2.reshape(1, -1)

    feats, idx = pl.pallas_call(
        _enc_argmin_kernel,
        grid=(NB,),
        in_specs=[
            pl.BlockSpec((BN, 8), lambda b: (b, 0)),
            pl.BlockSpec((8, M), lambda b: (0, 0)),
            pl.BlockSpec((8, H // 4), lambda b: (0, 0)),
            pl.BlockSpec((1, H // 4), lambda b: (0, 0)),
            pl.BlockSpec((H // 4, H // 2), lambda b: (0, 0)),
            pl.BlockSpec((1, H // 2), lambda b: (0, 0)),
            pl.BlockSpec((H // 2, H), lambda b: (0, 0)),
            pl.BlockSpec((1, H), lambda b: (0, 0)),
        ],
        out_specs=[
            pl.BlockSpec((BN, H), lambda b: (b, 0)),
            pl.BlockSpec((1, 1, BN), lambda b: (b, 0, 0)),
        ],
        out_shape=[
            jax.ShapeDtypeStruct((N, H), f32),
            jax.ShapeDtypeStruct((NB, 1, BN), jnp.int32),
        ],
        scratch_shapes=[pltpu.VMEM((8, M), jnp.bfloat16),
                        pltpu.VMEM((1, M), f32)],
    )(posP, gridP, W1P, b1r, W2, b2r, W3, b3r)

    wmask = pl.kernel(
        _winner_mask_sc,
        out_type=jax.ShapeDtypeStruct((N,), f32),
        mesh=plsc.VectorSubcoreMesh(core_axis_name="c", subcore_axis_name="s",
                                    num_cores=1),
        compiler_params=pltpu.CompilerParams(needs_layout_passes=False),
        scratch_types=[
            pltpu.VMEM((APW,), jnp.int32),
            pltpu.VMEM((M,), jnp.int32),
            pltpu.VMEM((SC_NS, CW), jnp.int32),
            pltpu.VMEM((CW,), jnp.int32),
            pltpu.VMEM((M,), jnp.int32),
            pltpu.VMEM((APW,), f32),
            pltpu.VMEM_SHARED((SC_NS, M), jnp.int32),
            pltpu.VMEM_SHARED((M,), jnp.int32),
            pltpu.SemaphoreType.DMA,
        ],
    )(idx.reshape(N))

    out = pl.pallas_call(
        _reduce_kernel,
        grid=(NB,),
        in_specs=[
            pl.BlockSpec((BN, H), lambda b: (b, 0)),
            pl.BlockSpec((1, 1, BN), lambda b: (b, 0, 0)),
            pl.BlockSpec((H, H), lambda b: (0, 0)),
            pl.BlockSpec((1, H), lambda b: (0, 0)),
            pl.BlockSpec((H, H), lambda b: (0, 0)),
            pl.BlockSpec((1, H), lambda b: (0, 0)),
            pl.BlockSpec((H, 4 * (H // 2)), lambda b: (0, 0)),
            pl.BlockSpec((1, 4 * (H // 2)), lambda b: (0, 0)),
            pl.BlockSpec((1, 4 * (H // 2)), lambda b: (0, 0)),
            pl.BlockSpec((1, 4), lambda b: (0, 0)),
        ],
        out_specs=pl.BlockSpec((1, 4), lambda b: (0, 0)),
        out_shape=jax.ShapeDtypeStruct((1, 4), f32),
        scratch_shapes=[pltpu.VMEM((1, H), f32), pltpu.VMEM((1, 1), f32)],
    )(feats, wmask.reshape(NB, 1, BN), Wn1, bn1r, Wn2, bn2r,
      Wh1r, bh1r, Wh2r, bh2r)

    return out.reshape(4)


# speedup vs baseline: 1.0787x; 1.0312x over previous
import jax
import jax.numpy as jnp
from jax import lax
from jax.experimental import pallas as pl
from jax.experimental.pallas import tpu as pltpu
from jax.experimental.pallas import tpu_sc as plsc

N = 4096
M = 8192
H = 256
BN = 512
NB = N // BN
CM = 2048
NC = M // CM

SC_NS = 16
APW = N // SC_NS
CW = M // SC_NS


def _bdot(a, b):
    return jnp.dot(a.astype(jnp.bfloat16), b.astype(jnp.bfloat16),
                   preferred_element_type=jnp.float32)


def _argmin_kernel(posP_ref, gridP_ref, idx_ref, gB_ref, gsq_ref):
    b = pl.program_id(0)

    @pl.when(b == 0)
    def _build_grid_aug():
        g = gridP_ref[...]
        gB_ref[...] = g.astype(jnp.bfloat16)
        gsq_ref[...] = ((g[0:1] * g[0:1] + g[1:2] * g[1:2])
                        + g[2:3] * g[2:3])

    p = posP_ref[...]
    asq = ((p[:, 0:1] * p[:, 0:1] + p[:, 1:2] * p[:, 1:2])
           + p[:, 2:3] * p[:, 2:3])
    pm2_bf = (p * -2.0).astype(jnp.bfloat16)
    gi_f = jax.lax.broadcasted_iota(jnp.int32, (BN, CM), 1).astype(jnp.float32)
    run_min = jnp.full((BN, 1), jnp.inf, jnp.float32)
    run_arg = jnp.zeros((BN, 1), jnp.float32)
    for c in range(NC):
        sl = slice(c * CM, (c + 1) * CM)
        ag2 = jnp.dot(pm2_bf, gB_ref[:, sl],
                      preferred_element_type=jnp.float32)
        d2 = (asq + ag2) + gsq_ref[:, sl]
        mn = jnp.min(d2, axis=1, keepdims=True)
        am = jnp.min(jnp.where(d2 == mn, gi_f, 3e9), axis=1,
                     keepdims=True) + (c * CM)
        upd = mn < run_min
        run_arg = jnp.where(upd, am, run_arg)
        run_min = jnp.minimum(run_min, mn)
    idx_ref[...] = jnp.swapaxes(run_arg.astype(jnp.int32), 0, 1)[None]


def _encz_kernel(posP_ref, W1_ref, b1_ref, W2_ref, b2_ref, W3_ref, b3_ref,
                 Wn1_ref, bn1_ref, z_ref):
    p = posP_ref[...]
    h = jnp.maximum(_bdot(p, W1_ref[...]) + b1_ref[...], 0.0)
    h = jnp.maximum(_bdot(h, W2_ref[...]) + b2_ref[...], 0.0)
    feats = _bdot(h, W3_ref[...]) + b3_ref[...]
    z_ref[...] = jnp.maximum(_bdot(feats, Wn1_ref[...]) + bn1_ref[...], 0.0)


def _winner_mask_sc(idx_hbm, w_hbm, idx_v, slot_v, tbuf_v, win_v, wfull_v,
                    wout_v, shared_slots, winner_sh, sem):
    wid = lax.axis_index("s")
    base_a = wid * APW
    pltpu.sync_copy(idx_hbm.at[pl.ds(base_a, APW)], idx_v)
    lane = lax.iota(jnp.int32, 16)

    neg1 = jnp.full((16,), -1, jnp.int32)
    for i in range(M // 16):
        slot_v[i * 16:(i + 1) * 16] = neg1

    for g in range(APW // 16):
        iv = idx_v[pl.ds(g * 16, 16)]
        val = base_a + g * 16 + lane
        for kk in range(16):
            plsc.store_scatter(slot_v, [iv], val, mask=(lane == kk))

    pltpu.sync_copy(slot_v, shared_slots.at[wid])
    plsc.subcore_barrier()

    base_c = wid * CW
    cps = [pltpu.async_copy(shared_slots.at[t, pl.ds(base_c, CW)],
                            tbuf_v.at[t], sem) for t in range(SC_NS)]
    for cp in cps:
        cp.wait()
    for cc in range(CW // 16):
        sl2 = pl.ds(cc * 16, 16)
        acc16 = tbuf_v[0, cc * 16:(cc + 1) * 16]
        for t in range(1, SC_NS):
            acc16 = jnp.maximum(acc16, tbuf_v[t, cc * 16:(cc + 1) * 16])
        win_v[sl2] = acc16
    pltpu.sync_copy(win_v, winner_sh.at[pl.ds(base_c, CW)])
    plsc.subcore_barrier()

    pltpu.sync_copy(winner_sh, wfull_v)
    for g in range(APW // 16):
        iv = idx_v[pl.ds(g * 16, 16)]
        wn = plsc.load_gather(wfull_v, [iv])
        myid = base_a + g * 16 + lane
        wout_v[pl.ds(g * 16, 16)] = jnp.where(wn == myid, 1.0, 0.0)
    pltpu.sync_copy(wout_v, w_hbm.at[pl.ds(base_a, APW)])


def _final_kernel(z_ref, w_ref, bn1_ref, Wn2_ref, bn2_ref,
                  Wh1_ref, bh1_ref, Wh2_ref, bh2_ref, out_ref):
    w = w_ref[...]
    S = jnp.dot(w, z_ref[...], preferred_element_type=jnp.float32,
                precision=jax.lax.Precision.HIGHEST)
    nocc = jnp.sum(w, axis=1, keepdims=True)
    srel = S + (M - nocc) * jnp.maximum(bn1_ref[...], 0.0)
    agg = _bdot(srel * (1.0 / M), Wn2_ref[...]) + bn2_ref[...]
    hh = jnp.maximum(_bdot(agg, Wh1_ref[...]) + bh1_ref[...], 0.0)
    prod = (hh.astype(jnp.bfloat16).astype(jnp.float32)
            * Wh2_ref[...].astype(jnp.bfloat16).astype(jnp.float32))
    hw = H // 2
    for k in range(4):
        sv = jnp.sum(prod[:, k * hw:(k + 1) * hw], axis=1, keepdims=True)
        out_ref[:, k:k + 1] = sv + bh2_ref[:, k:k + 1]


def kernel(atomic_positions, grid_points, W1, b1, W2, b2, W3, b3,
           Wn1, bn1, Wn2, bn2, Wh1, bh1, Wh2, bh2):
    f32 = jnp.float32
    posP = jnp.pad(atomic_positions, ((0, 0), (0, 5)))
    gridP = jnp.pad(grid_points.T, ((0, 5), (0, 0)))
    W1P = jnp.pad(W1, ((0, 5), (0, 0)))
    b1r = b1.reshape(1, -1)
    b2r = b2.reshape(1, -1)
    b3r = b3.reshape(1, -1)
    bn1r = bn1.reshape(1, -1)
    bn2r = bn2.reshape(1, -1)
    Wh1r = Wh1.transpose(1, 0, 2).reshape(H, 4 * (H // 2))
    bh1r = bh1.reshape(1, -1)
    Wh2r = Wh2[:, :, 0].reshape(1, -1)
    bh2r = bh2.reshape(1, -1)

    idx = pl.pallas_call(
        _argmin_kernel,
        grid=(NB,),
        in_specs=[
            pl.BlockSpec((BN, 8), lambda b: (b, 0)),
            pl.BlockSpec((8, M), lambda b: (0, 0)),
        ],
        out_specs=pl.BlockSpec((1, 1, BN), lambda b: (b, 0, 0)),
        out_shape=jax.ShapeDtypeStruct((NB, 1, BN), jnp.int32),
        scratch_shapes=[pltpu.VMEM((8, M), jnp.bfloat16),
                        pltpu.VMEM((1, M), f32)],
    )(posP, gridP)

    wmask = pl.kernel(
        _winner_mask_sc,
        out_type=jax.ShapeDtypeStruct((N,), f32),
        mesh=plsc.VectorSubcoreMesh(core_axis_name="c", subcore_axis_name="s",
                                    num_cores=1),
        compiler_params=pltpu.CompilerParams(needs_layout_passes=False),
        scratch_types=[
            pltpu.VMEM((APW,), jnp.int32),
            pltpu.VMEM((M,), jnp.int32),
            pltpu.VMEM((SC_NS, CW), jnp.int32),
            pltpu.VMEM((CW,), jnp.int32),
            pltpu.VMEM((M,), jnp.int32),
            pltpu.VMEM((APW,), f32),
            pltpu.VMEM_SHARED((SC_NS, M), jnp.int32),
            pltpu.VMEM_SHARED((M,), jnp.int32),
            pltpu.SemaphoreType.DMA,
        ],
    )(idx.reshape(N))

    z = pl.pallas_call(
        _encz_kernel,
        grid=(NB,),
        in_specs=[
            pl.BlockSpec((BN, 8), lambda b: (b, 0)),
            pl.BlockSpec((8, H // 4), lambda b: (0, 0)),
            pl.BlockSpec((1, H // 4), lambda b: (0, 0)),
            pl.BlockSpec((H // 4, H // 2), lambda b: (0, 0)),
            pl.BlockSpec((1, H // 2), lambda b: (0, 0)),
            pl.BlockSpec((H // 2, H), lambda b: (0, 0)),
            pl.BlockSpec((1, H), lambda b: (0, 0)),
            pl.BlockSpec((H, H), lambda b: (0, 0)),
            pl.BlockSpec((1, H), lambda b: (0, 0)),
        ],
        out_specs=pl.BlockSpec((BN, H), lambda b: (b, 0)),
        out_shape=jax.ShapeDtypeStruct((N, H), f32),
    )(posP, W1P, b1r, W2, b2r, W3, b3r, Wn1, bn1r)

    out = pl.pallas_call(
        _final_kernel,
        grid=(1,),
        in_specs=[
            pl.BlockSpec((N, H), lambda b: (0, 0)),
            pl.BlockSpec((1, N), lambda b: (0, 0)),
            pl.BlockSpec((1, H), lambda b: (0, 0)),
            pl.BlockSpec((H, H), lambda b: (0, 0)),
            pl.BlockSpec((1, H), lambda b: (0, 0)),
            pl.BlockSpec((H, 4 * (H // 2)), lambda b: (0, 0)),
            pl.BlockSpec((1, 4 * (H // 2)), lambda b: (0, 0)),
            pl.BlockSpec((1, 4 * (H // 2)), lambda b: (0, 0)),
            pl.BlockSpec((1, 4), lambda b: (0, 0)),
        ],
        out_specs=pl.BlockSpec((1, 4), lambda b: (0, 0)),
        out_shape=jax.ShapeDtypeStruct((1, 4), f32),
    )(z, wmask.reshape(1, N), bn1r, Wn2, bn2r, Wh1r, bh1r, Wh2r, bh2r)

    return out.reshape(4)


# speedup vs baseline: 1.0989x; 1.0188x over previous
import jax
import jax.numpy as jnp
from jax import lax
from jax.experimental import pallas as pl
from jax.experimental.pallas import tpu as pltpu
from jax.experimental.pallas import tpu_sc as plsc

N = 4096
M = 8192
H = 256
BN = 512
NB = N // BN
CM = 2048
NC = M // CM

SC_NS = 16
APW = N // SC_NS
CW = M // SC_NS


def _bdot(a, b):
    return jnp.dot(a.astype(jnp.bfloat16), b.astype(jnp.bfloat16),
                   preferred_element_type=jnp.float32)


def _argmin_kernel(posP_ref, gridP_ref, idx_ref, gB_ref, gsq_ref):
    b = pl.program_id(0)

    @pl.when(b == 0)
    def _build_grid_aug():
        g = gridP_ref[...]
        gB_ref[...] = g.astype(jnp.bfloat16)
        gsq_ref[...] = ((g[0:1] * g[0:1] + g[1:2] * g[1:2])
                        + g[2:3] * g[2:3])

    p = posP_ref[...]
    asq = ((p[:, 0:1] * p[:, 0:1] + p[:, 1:2] * p[:, 1:2])
           + p[:, 2:3] * p[:, 2:3])
    pm2_bf = (p * -2.0).astype(jnp.bfloat16)
    gi_f = jax.lax.broadcasted_iota(jnp.int32, (BN, CM), 1).astype(jnp.float32)
    run_min = jnp.full((BN, 1), jnp.inf, jnp.float32)
    run_arg = jnp.zeros((BN, 1), jnp.float32)
    for c in range(NC):
        sl = slice(c * CM, (c + 1) * CM)
        ag2 = jnp.dot(pm2_bf, gB_ref[:, sl],
                      preferred_element_type=jnp.float32)
        d2 = (asq + ag2) + gsq_ref[:, sl]
        mn = jnp.min(d2, axis=1, keepdims=True)
        am = jnp.min(jnp.where(d2 == mn, gi_f, 3e9), axis=1,
                     keepdims=True) + (c * CM)
        upd = mn < run_min
        run_arg = jnp.where(upd, am, run_arg)
        run_min = jnp.minimum(run_min, mn)
    idx_ref[...] = jnp.swapaxes(run_arg.astype(jnp.int32), 0, 1)[None]


def _encz_kernel(posP_ref, W1_ref, b1_ref, W2_ref, b2_ref, W3_ref, b3_ref,
                 Wn1_ref, bn1_ref, z_ref):
    p = posP_ref[...]
    h = jnp.maximum(_bdot(p, W1_ref[...]) + b1_ref[...], 0.0)
    h = jnp.maximum(_bdot(h, W2_ref[...]) + b2_ref[...], 0.0)
    feats = _bdot(h, W3_ref[...]) + b3_ref[...]
    z_ref[...] = jnp.maximum(_bdot(feats, Wn1_ref[...]) + bn1_ref[...], 0.0)


def _winner_mask_sc(idx_hbm, w_hbm, idx_v, slot_v, tbuf_v, win_v, wfull_v,
                    wout_v, shared_slots, winner_sh, sem):
    wid = lax.axis_index("s")
    base_a = wid * APW
    pltpu.sync_copy(idx_hbm.at[pl.ds(base_a, APW)], idx_v)
    lane = lax.iota(jnp.int32, 16)

    neg1 = jnp.full((16,), -1, jnp.int32)
    for i in range(M // 16):
        slot_v[i * 16:(i + 1) * 16] = neg1

    for g in range(APW // 16):
        iv = idx_v[pl.ds(g * 16, 16)]
        val = base_a + g * 16 + lane
        for kk in range(16):
            plsc.store_scatter(slot_v, [iv], val, mask=(lane == kk))

    pltpu.sync_copy(slot_v, shared_slots.at[wid])
    plsc.subcore_barrier()

    base_c = wid * CW
    cps = [pltpu.async_copy(shared_slots.at[t, pl.ds(base_c, CW)],
                            tbuf_v.at[t], sem) for t in range(SC_NS)]
    for cp in cps:
        cp.wait()
    for cc in range(CW // 16):
        sl2 = pl.ds(cc * 16, 16)
        acc16 = tbuf_v[0, cc * 16:(cc + 1) * 16]
        for t in range(1, SC_NS):
            acc16 = jnp.maximum(acc16, tbuf_v[t, cc * 16:(cc + 1) * 16])
        win_v[sl2] = acc16
    pltpu.sync_copy(win_v, winner_sh.at[pl.ds(base_c, CW)])
    plsc.subcore_barrier()

    pltpu.sync_copy(winner_sh, wfull_v)
    for g in range(APW // 16):
        iv = idx_v[pl.ds(g * 16, 16)]
        wn = plsc.load_gather(wfull_v, [iv])
        myid = base_a + g * 16 + lane
        wout_v[pl.ds(g * 16, 16)] = jnp.where(wn == myid, 1.0, 0.0)
    pltpu.sync_copy(wout_v, w_hbm.at[pl.ds(base_a, APW)])


def _final_kernel(z_ref, w_ref, bn1_ref, Wn2_ref, bn2_ref,
                  Wh1_ref, bh1_ref, Wh2_ref, bh2_ref, out_ref):
    w = w_ref[...]
    S = _bdot(w, z_ref[...])
    nocc = jnp.sum(w, axis=1, keepdims=True)
    bn1r = jnp.maximum(bn1_ref[...], 0.0)
    bn1rb = bn1r.astype(jnp.bfloat16).astype(jnp.float32)
    srel = S + (M - nocc) * bn1rb
    Wn2b = Wn2_ref[...].astype(jnp.bfloat16).astype(jnp.float32)
    agg = (jnp.dot(srel, Wn2b, preferred_element_type=jnp.float32,
                   precision=jax.lax.Precision.HIGHEST) * (1.0 / M)
           + bn2_ref[...])
    hh = jnp.maximum(_bdot(agg, Wh1_ref[...]) + bh1_ref[...], 0.0)
    prod = (hh.astype(jnp.bfloat16).astype(jnp.float32)
            * Wh2_ref[...].astype(jnp.bfloat16).astype(jnp.float32))
    hw = H // 2
    for k in range(4):
        sv = jnp.sum(prod[:, k * hw:(k + 1) * hw], axis=1, keepdims=True)
        out_ref[:, k:k + 1] = sv + bh2_ref[:, k:k + 1]


def kernel(atomic_positions, grid_points, W1, b1, W2, b2, W3, b3,
           Wn1, bn1, Wn2, bn2, Wh1, bh1, Wh2, bh2):
    f32 = jnp.float32
    posP = jnp.pad(atomic_positions, ((0, 0), (0, 5)))
    gridP = jnp.pad(grid_points.T, ((0, 5), (0, 0)))
    W1P = jnp.pad(W1, ((0, 5), (0, 0)))
    b1r = b1.reshape(1, -1)
    b2r = b2.reshape(1, -1)
    b3r = b3.reshape(1, -1)
    bn1r = bn1.reshape(1, -1)
    bn2r = bn2.reshape(1, -1)
    Wh1r = Wh1.transpose(1, 0, 2).reshape(H, 4 * (H // 2))
    bh1r = bh1.reshape(1, -1)
    Wh2r = Wh2[:, :, 0].reshape(1, -1)
    bh2r = bh2.reshape(1, -1)

    idx = pl.pallas_call(
        _argmin_kernel,
        grid=(NB,),
        in_specs=[
            pl.BlockSpec((BN, 8), lambda b: (b, 0)),
            pl.BlockSpec((8, M), lambda b: (0, 0)),
        ],
        out_specs=pl.BlockSpec((1, 1, BN), lambda b: (b, 0, 0)),
        out_shape=jax.ShapeDtypeStruct((NB, 1, BN), jnp.int32),
        scratch_shapes=[pltpu.VMEM((8, M), jnp.bfloat16),
                        pltpu.VMEM((1, M), f32)],
    )(posP, gridP)

    wmask = pl.kernel(
        _winner_mask_sc,
        out_type=jax.ShapeDtypeStruct((N,), f32),
        mesh=plsc.VectorSubcoreMesh(core_axis_name="c", subcore_axis_name="s",
                                    num_cores=1),
        compiler_params=pltpu.CompilerParams(needs_layout_passes=False),
        scratch_types=[
            pltpu.VMEM((APW,), jnp.int32),
            pltpu.VMEM((M,), jnp.int32),
            pltpu.VMEM((SC_NS, CW), jnp.int32),
            pltpu.VMEM((CW,), jnp.int32),
            pltpu.VMEM((M,), jnp.int32),
            pltpu.VMEM((APW,), f32),
            pltpu.VMEM_SHARED((SC_NS, M), jnp.int32),
            pltpu.VMEM_SHARED((M,), jnp.int32),
            pltpu.SemaphoreType.DMA,
        ],
    )(idx.reshape(N))

    z = pl.pallas_call(
        _encz_kernel,
        grid=(NB,),
        in_specs=[
            pl.BlockSpec((BN, 8), lambda b: (b, 0)),
            pl.BlockSpec((8, H // 4), lambda b: (0, 0)),
            pl.BlockSpec((1, H // 4), lambda b: (0, 0)),
            pl.BlockSpec((H // 4, H // 2), lambda b: (0, 0)),
            pl.BlockSpec((1, H // 2), lambda b: (0, 0)),
            pl.BlockSpec((H // 2, H), lambda b: (0, 0)),
            pl.BlockSpec((1, H), lambda b: (0, 0)),
            pl.BlockSpec((H, H), lambda b: (0, 0)),
            pl.BlockSpec((1, H), lambda b: (0, 0)),
        ],
        out_specs=pl.BlockSpec((BN, H), lambda b: (b, 0)),
        out_shape=jax.ShapeDtypeStruct((N, H), f32),
    )(posP, W1P, b1r, W2, b2r, W3, b3r, Wn1, bn1r)

    out = pl.pallas_call(
        _final_kernel,
        grid=(1,),
        in_specs=[
            pl.BlockSpec((N, H), lambda b: (0, 0)),
            pl.BlockSpec((1, N), lambda b: (0, 0)),
            pl.BlockSpec((1, H), lambda b: (0, 0)),
            pl.BlockSpec((H, H), lambda b: (0, 0)),
            pl.BlockSpec((1, H), lambda b: (0, 0)),
            pl.BlockSpec((H, 4 * (H // 2)), lambda b: (0, 0)),
            pl.BlockSpec((1, 4 * (H // 2)), lambda b: (0, 0)),
            pl.BlockSpec((1, 4 * (H // 2)), lambda b: (0, 0)),
            pl.BlockSpec((1, 4), lambda b: (0, 0)),
        ],
        out_specs=pl.BlockSpec((1, 4), lambda b: (0, 0)),
        out_shape=jax.ShapeDtypeStruct((1, 4), f32),
    )(z, wmask.reshape(1, N), bn1r, Wn2, bn2r, Wh1r, bh1r, Wh2r, bh2r)

    return out.reshape(4)


# speedup vs baseline: 1.1727x; 1.0671x over previous
import jax
import jax.numpy as jnp
from jax import lax
from jax.experimental import pallas as pl
from jax.experimental.pallas import tpu as pltpu
from jax.experimental.pallas import tpu_sc as plsc

N = 4096
M = 8192
H = 256
BN = 1024
NB = N // BN
CM = 2048
NC = M // CM

SC_NS = 16
APW = N // SC_NS
CW = M // SC_NS


def _bdot(a, b):
    return jnp.dot(a.astype(jnp.bfloat16), b.astype(jnp.bfloat16),
                   preferred_element_type=jnp.float32)


def _argmin_kernel(posP_ref, gridP_ref, idx_ref, gB_ref, gsq_ref):
    b = pl.program_id(0)

    @pl.when(b == 0)
    def _build_grid_aug():
        g = gridP_ref[...]
        gB_ref[...] = g.astype(jnp.bfloat16)
        gsq_ref[...] = ((g[0:1] * g[0:1] + g[1:2] * g[1:2])
                        + g[2:3] * g[2:3])

    p = posP_ref[...]
    asq = ((p[:, 0:1] * p[:, 0:1] + p[:, 1:2] * p[:, 1:2])
           + p[:, 2:3] * p[:, 2:3])
    pm2_bf = (p * -2.0).astype(jnp.bfloat16)
    gi_f = jax.lax.broadcasted_iota(jnp.int32, (BN, CM), 1).astype(jnp.float32)
    run_min = jnp.full((BN, 1), jnp.inf, jnp.float32)
    run_arg = jnp.zeros((BN, 1), jnp.float32)
    for c in range(NC):
        sl = slice(c * CM, (c + 1) * CM)
        ag2 = jnp.dot(pm2_bf, gB_ref[:, sl],
                      preferred_element_type=jnp.float32)
        d2 = (asq + ag2) + gsq_ref[:, sl]
        mn = jnp.min(d2, axis=1, keepdims=True)
        am = jnp.min(jnp.where(d2 == mn, gi_f, 3e9), axis=1,
                     keepdims=True) + (c * CM)
        upd = mn < run_min
        run_arg = jnp.where(upd, am, run_arg)
        run_min = jnp.minimum(run_min, mn)
    idx_ref[...] = jnp.swapaxes(run_arg.astype(jnp.int32), 0, 1)[None]


def _encz_kernel(posP_ref, W1_ref, b1_ref, W2_ref, b2_ref, W3_ref, b3_ref,
                 Wn1_ref, bn1_ref, z_ref):
    p = posP_ref[...]
    h = jnp.maximum(_bdot(p, W1_ref[...]) + b1_ref[...], 0.0)
    h = jnp.maximum(_bdot(h, W2_ref[...]) + b2_ref[...], 0.0)
    feats = _bdot(h, W3_ref[...]) + b3_ref[...]
    z_ref[...] = jnp.maximum(_bdot(feats, Wn1_ref[...]) + bn1_ref[...], 0.0)


def _winner_mask_sc(idx_hbm, w_hbm, idx_v, slot_v, tbuf_v, win_v, wfull_v,
                    wout_v, shared_slots, winner_sh, sem):
    wid = lax.axis_index("s")
    base_a = wid * APW
    pltpu.sync_copy(idx_hbm.at[pl.ds(base_a, APW)], idx_v)
    lane = lax.iota(jnp.int32, 16)

    neg1 = jnp.full((16,), -1, jnp.int32)
    for i in range(M // 16):
        slot_v[i * 16:(i + 1) * 16] = neg1

    for g in range(APW // 16):
        iv = idx_v[pl.ds(g * 16, 16)]
        val = base_a + g * 16 + lane
        for kk in range(16):
            plsc.store_scatter(slot_v, [iv], val, mask=(lane == kk))

    pltpu.sync_copy(slot_v, shared_slots.at[wid])
    plsc.subcore_barrier()

    base_c = wid * CW
    cps = [pltpu.async_copy(shared_slots.at[t, pl.ds(base_c, CW)],
                            tbuf_v.at[t], sem) for t in range(SC_NS)]
    for cp in cps:
        cp.wait()
    for cc in range(CW // 16):
        sl2 = pl.ds(cc * 16, 16)
        acc16 = tbuf_v[0, cc * 16:(cc + 1) * 16]
        for t in range(1, SC_NS):
            acc16 = jnp.maximum(acc16, tbuf_v[t, cc * 16:(cc + 1) * 16])
        win_v[sl2] = acc16
    pltpu.sync_copy(win_v, winner_sh.at[pl.ds(base_c, CW)])
    plsc.subcore_barrier()

    pltpu.sync_copy(winner_sh, wfull_v)
    for g in range(APW // 16):
        iv = idx_v[pl.ds(g * 16, 16)]
        wn = plsc.load_gather(wfull_v, [iv])
        myid = base_a + g * 16 + lane
        wout_v[pl.ds(g * 16, 16)] = jnp.where(wn == myid, 1.0, 0.0)
    pltpu.sync_copy(wout_v, w_hbm.at[pl.ds(base_a, APW)])


def _final_kernel(z_ref, w_ref, bn1_ref, Wn2_ref, bn2_ref,
                  Wh1_ref, bh1_ref, Wh2_ref, bh2_ref, out_ref):
    w = w_ref[...]
    S = _bdot(w, z_ref[...])
    nocc = jnp.sum(w, axis=1, keepdims=True)
    bn1r = jnp.maximum(bn1_ref[...], 0.0)
    bn1rb = bn1r.astype(jnp.bfloat16).astype(jnp.float32)
    srel = S + (M - nocc) * bn1rb
    Wn2b = Wn2_ref[...].astype(jnp.bfloat16).astype(jnp.float32)
    agg = (jnp.dot(srel, Wn2b, preferred_element_type=jnp.float32,
                   precision=jax.lax.Precision.HIGHEST) * (1.0 / M)
           + bn2_ref[...])
    hh = jnp.maximum(_bdot(agg, Wh1_ref[...]) + bh1_ref[...], 0.0)
    prod = (hh.astype(jnp.bfloat16).astype(jnp.float32)
            * Wh2_ref[...].astype(jnp.bfloat16).astype(jnp.float32))
    hw = H // 2
    for k in range(4):
        sv = jnp.sum(prod[:, k * hw:(k + 1) * hw], axis=1, keepdims=True)
        out_ref[:, k:k + 1] = sv + bh2_ref[:, k:k + 1]


def kernel(atomic_positions, grid_points, W1, b1, W2, b2, W3, b3,
           Wn1, bn1, Wn2, bn2, Wh1, bh1, Wh2, bh2):
    f32 = jnp.float32
    posP = jnp.pad(atomic_positions, ((0, 0), (0, 5)))
    gridP = jnp.pad(grid_points.T, ((0, 5), (0, 0)))
    W1P = jnp.pad(W1, ((0, 5), (0, 0)))
    b1r = b1.reshape(1, -1)
    b2r = b2.reshape(1, -1)
    b3r = b3.reshape(1, -1)
    bn1r = bn1.reshape(1, -1)
    bn2r = bn2.reshape(1, -1)
    Wh1r = Wh1.transpose(1, 0, 2).reshape(H, 4 * (H // 2))
    bh1r = bh1.reshape(1, -1)
    Wh2r = Wh2[:, :, 0].reshape(1, -1)
    bh2r = bh2.reshape(1, -1)

    idx = pl.pallas_call(
        _argmin_kernel,
        grid=(NB,),
        in_specs=[
            pl.BlockSpec((BN, 8), lambda b: (b, 0)),
            pl.BlockSpec((8, M), lambda b: (0, 0)),
        ],
        out_specs=pl.BlockSpec((1, 1, BN), lambda b: (b, 0, 0)),
        out_shape=jax.ShapeDtypeStruct((NB, 1, BN), jnp.int32),
        scratch_shapes=[pltpu.VMEM((8, M), jnp.bfloat16),
                        pltpu.VMEM((1, M), f32)],
    )(posP, gridP)

    wmask = pl.kernel(
        _winner_mask_sc,
        out_type=jax.ShapeDtypeStruct((N,), f32),
        mesh=plsc.VectorSubcoreMesh(core_axis_name="c", subcore_axis_name="s",
                                    num_cores=1),
        compiler_params=pltpu.CompilerParams(needs_layout_passes=False),
        scratch_types=[
            pltpu.VMEM((APW,), jnp.int32),
            pltpu.VMEM((M,), jnp.int32),
            pltpu.VMEM((SC_NS, CW), jnp.int32),
            pltpu.VMEM((CW,), jnp.int32),
            pltpu.VMEM((M,), jnp.int32),
            pltpu.VMEM((APW,), f32),
            pltpu.VMEM_SHARED((SC_NS, M), jnp.int32),
            pltpu.VMEM_SHARED((M,), jnp.int32),
            pltpu.SemaphoreType.DMA,
        ],
    )(idx.reshape(N))

    z = pl.pallas_call(
        _encz_kernel,
        grid=(NB,),
        in_specs=[
            pl.BlockSpec((BN, 8), lambda b: (b, 0)),
            pl.BlockSpec((8, H // 4), lambda b: (0, 0)),
            pl.BlockSpec((1, H // 4), lambda b: (0, 0)),
            pl.BlockSpec((H // 4, H // 2), lambda b: (0, 0)),
            pl.BlockSpec((1, H // 2), lambda b: (0, 0)),
            pl.BlockSpec((H // 2, H), lambda b: (0, 0)),
            pl.BlockSpec((1, H), lambda b: (0, 0)),
            pl.BlockSpec((H, H), lambda b: (0, 0)),
            pl.BlockSpec((1, H), lambda b: (0, 0)),
        ],
        out_specs=pl.BlockSpec((BN, H), lambda b: (b, 0)),
        out_shape=jax.ShapeDtypeStruct((N, H), f32),
    )(posP, W1P, b1r, W2, b2r, W3, b3r, Wn1, bn1r)

    out = pl.pallas_call(
        _final_kernel,
        grid=(1,),
        in_specs=[
            pl.BlockSpec((N, H), lambda b: (0, 0)),
            pl.BlockSpec((1, N), lambda b: (0, 0)),
            pl.BlockSpec((1, H), lambda b: (0, 0)),
            pl.BlockSpec((H, H), lambda b: (0, 0)),
            pl.BlockSpec((1, H), lambda b: (0, 0)),
            pl.BlockSpec((H, 4 * (H // 2)), lambda b: (0, 0)),
            pl.BlockSpec((1, 4 * (H // 2)), lambda b: (0, 0)),
            pl.BlockSpec((1, 4 * (H // 2)), lambda b: (0, 0)),
            pl.BlockSpec((1, 4), lambda b: (0, 0)),
        ],
        out_specs=pl.BlockSpec((1, 4), lambda b: (0, 0)),
        out_shape=jax.ShapeDtypeStruct((1, 4), f32),
    )(z, wmask.reshape(1, N), bn1r, Wn2, bn2r, Wh1r, bh1r, Wh2r, bh2r)

    return out.reshape(4)


# speedup vs baseline: 1.1762x; 1.0030x over previous
import jax
import jax.numpy as jnp
from jax import lax
from jax.experimental import pallas as pl
from jax.experimental.pallas import tpu as pltpu
from jax.experimental.pallas import tpu_sc as plsc

N = 4096
M = 8192
H = 256
BN = 2048
NB = N // BN
CM = 2048
NC = M // CM

SC_NS = 16
APW = N // SC_NS
CW = M // SC_NS


def _bdot(a, b):
    return jnp.dot(a.astype(jnp.bfloat16), b.astype(jnp.bfloat16),
                   preferred_element_type=jnp.float32)


def _argmin_kernel(posP_ref, gridP_ref, idx_ref, gB_ref, gsq_ref):
    b = pl.program_id(0)

    @pl.when(b == 0)
    def _build_grid_aug():
        g = gridP_ref[...]
        gB_ref[...] = g.astype(jnp.bfloat16)
        gsq_ref[...] = ((g[0:1] * g[0:1] + g[1:2] * g[1:2])
                        + g[2:3] * g[2:3])

    p = posP_ref[...]
    asq = ((p[:, 0:1] * p[:, 0:1] + p[:, 1:2] * p[:, 1:2])
           + p[:, 2:3] * p[:, 2:3])
    pm2_bf = (p * -2.0).astype(jnp.bfloat16)
    gi_f = jax.lax.broadcasted_iota(jnp.int32, (BN, CM), 1).astype(jnp.float32)
    run_min = jnp.full((BN, 1), jnp.inf, jnp.float32)
    run_arg = jnp.zeros((BN, 1), jnp.float32)
    for c in range(NC):
        sl = slice(c * CM, (c + 1) * CM)
        ag2 = jnp.dot(pm2_bf, gB_ref[:, sl],
                      preferred_element_type=jnp.float32)
        d2 = (asq + ag2) + gsq_ref[:, sl]
        mn = jnp.min(d2, axis=1, keepdims=True)
        am = jnp.min(jnp.where(d2 == mn, gi_f, 3e9), axis=1,
                     keepdims=True) + (c * CM)
        upd = mn < run_min
        run_arg = jnp.where(upd, am, run_arg)
        run_min = jnp.minimum(run_min, mn)
    idx_ref[...] = jnp.swapaxes(run_arg.astype(jnp.int32), 0, 1)[None]


def _encz_kernel(posP_ref, W1_ref, b1_ref, W2_ref, b2_ref, W3_ref, b3_ref,
                 Wn1_ref, bn1_ref, z_ref):
    p = posP_ref[...]
    h = jnp.maximum(_bdot(p, W1_ref[...]) + b1_ref[...], 0.0)
    h = jnp.maximum(_bdot(h, W2_ref[...]) + b2_ref[...], 0.0)
    feats = _bdot(h, W3_ref[...]) + b3_ref[...]
    z_ref[...] = jnp.maximum(_bdot(feats, Wn1_ref[...]) + bn1_ref[...], 0.0)


def _winner_mask_sc(idx_hbm, w_hbm, idx_v, slot_v, tbuf_v, win_v, wfull_v,
                    wout_v, shared_slots, winner_sh, sem):
    wid = lax.axis_index("s")
    base_a = wid * APW
    pltpu.sync_copy(idx_hbm.at[pl.ds(base_a, APW)], idx_v)
    lane = lax.iota(jnp.int32, 16)

    neg1 = jnp.full((16,), -1, jnp.int32)
    for i in range(M // 16):
        slot_v[i * 16:(i + 1) * 16] = neg1

    for g in range(APW // 16):
        iv = idx_v[pl.ds(g * 16, 16)]
        val = base_a + g * 16 + lane
        for kk in range(16):
            plsc.store_scatter(slot_v, [iv], val, mask=(lane == kk))

    pltpu.sync_copy(slot_v, shared_slots.at[wid])
    plsc.subcore_barrier()

    base_c = wid * CW
    cps = [pltpu.async_copy(shared_slots.at[t, pl.ds(base_c, CW)],
                            tbuf_v.at[t], sem) for t in range(SC_NS)]
    for cp in cps:
        cp.wait()
    for cc in range(CW // 16):
        sl2 = pl.ds(cc * 16, 16)
        acc16 = tbuf_v[0, cc * 16:(cc + 1) * 16]
        for t in range(1, SC_NS):
            acc16 = jnp.maximum(acc16, tbuf_v[t, cc * 16:(cc + 1) * 16])
        win_v[sl2] = acc16
    pltpu.sync_copy(win_v, winner_sh.at[pl.ds(base_c, CW)])
    plsc.subcore_barrier()

    pltpu.sync_copy(winner_sh, wfull_v)
    for g in range(APW // 16):
        iv = idx_v[pl.ds(g * 16, 16)]
        wn = plsc.load_gather(wfull_v, [iv])
        myid = base_a + g * 16 + lane
        wout_v[pl.ds(g * 16, 16)] = jnp.where(wn == myid, 1.0, 0.0)
    pltpu.sync_copy(wout_v, w_hbm.at[pl.ds(base_a, APW)])


def _final_kernel(z_ref, w_ref, bn1_ref, Wn2_ref, bn2_ref,
                  Wh1_ref, bh1_ref, Wh2_ref, bh2_ref, out_ref):
    w = w_ref[...]
    S = _bdot(w, z_ref[...])
    nocc = jnp.sum(w, axis=1, keepdims=True)
    bn1r = jnp.maximum(bn1_ref[...], 0.0)
    bn1rb = bn1r.astype(jnp.bfloat16).astype(jnp.float32)
    srel = S + (M - nocc) * bn1rb
    Wn2b = Wn2_ref[...].astype(jnp.bfloat16).astype(jnp.float32)
    agg = (jnp.dot(srel, Wn2b, preferred_element_type=jnp.float32,
                   precision=jax.lax.Precision.HIGHEST) * (1.0 / M)
           + bn2_ref[...])
    hh = jnp.maximum(_bdot(agg, Wh1_ref[...]) + bh1_ref[...], 0.0)
    prod = (hh.astype(jnp.bfloat16).astype(jnp.float32)
            * Wh2_ref[...].astype(jnp.bfloat16).astype(jnp.float32))
    hw = H // 2
    for k in range(4):
        sv = jnp.sum(prod[:, k * hw:(k + 1) * hw], axis=1, keepdims=True)
        out_ref[:, k:k + 1] = sv + bh2_ref[:, k:k + 1]


def kernel(atomic_positions, grid_points, W1, b1, W2, b2, W3, b3,
           Wn1, bn1, Wn2, bn2, Wh1, bh1, Wh2, bh2):
    f32 = jnp.float32
    posP = jnp.pad(atomic_positions, ((0, 0), (0, 5)))
    gridP = jnp.pad(grid_points.T, ((0, 5), (0, 0)))
    W1P = jnp.pad(W1, ((0, 5), (0, 0)))
    b1r = b1.reshape(1, -1)
    b2r = b2.reshape(1, -1)
    b3r = b3.reshape(1, -1)
    bn1r = bn1.reshape(1, -1)
    bn2r = bn2.reshape(1, -1)
    Wh1r = Wh1.transpose(1, 0, 2).reshape(H, 4 * (H // 2))
    bh1r = bh1.reshape(1, -1)
    Wh2r = Wh2[:, :, 0].reshape(1, -1)
    bh2r = bh2.reshape(1, -1)

    idx = pl.pallas_call(
        _argmin_kernel,
        grid=(NB,),
        in_specs=[
            pl.BlockSpec((BN, 8), lambda b: (b, 0)),
            pl.BlockSpec((8, M), lambda b: (0, 0)),
        ],
        out_specs=pl.BlockSpec((1, 1, BN), lambda b: (b, 0, 0)),
        out_shape=jax.ShapeDtypeStruct((NB, 1, BN), jnp.int32),
        scratch_shapes=[pltpu.VMEM((8, M), jnp.bfloat16),
                        pltpu.VMEM((1, M), f32)],
    )(posP, gridP)

    wmask = pl.kernel(
        _winner_mask_sc,
        out_type=jax.ShapeDtypeStruct((N,), f32),
        mesh=plsc.VectorSubcoreMesh(core_axis_name="c", subcore_axis_name="s",
                                    num_cores=1),
        compiler_params=pltpu.CompilerParams(needs_layout_passes=False),
        scratch_types=[
            pltpu.VMEM((APW,), jnp.int32),
            pltpu.VMEM((M,), jnp.int32),
            pltpu.VMEM((SC_NS, CW), jnp.int32),
            pltpu.VMEM((CW,), jnp.int32),
            pltpu.VMEM((M,), jnp.int32),
            pltpu.VMEM((APW,), f32),
            pltpu.VMEM_SHARED((SC_NS, M), jnp.int32),
            pltpu.VMEM_SHARED((M,), jnp.int32),
            pltpu.SemaphoreType.DMA,
        ],
    )(idx.reshape(N))

    z = pl.pallas_call(
        _encz_kernel,
        grid=(NB,),
        in_specs=[
            pl.BlockSpec((BN, 8), lambda b: (b, 0)),
            pl.BlockSpec((8, H // 4), lambda b: (0, 0)),
            pl.BlockSpec((1, H // 4), lambda b: (0, 0)),
            pl.BlockSpec((H // 4, H // 2), lambda b: (0, 0)),
            pl.BlockSpec((1, H // 2), lambda b: (0, 0)),
            pl.BlockSpec((H // 2, H), lambda b: (0, 0)),
            pl.BlockSpec((1, H), lambda b: (0, 0)),
            pl.BlockSpec((H, H), lambda b: (0, 0)),
            pl.BlockSpec((1, H), lambda b: (0, 0)),
        ],
        out_specs=pl.BlockSpec((BN, H), lambda b: (b, 0)),
        out_shape=jax.ShapeDtypeStruct((N, H), f32),
    )(posP, W1P, b1r, W2, b2r, W3, b3r, Wn1, bn1r)

    out = pl.pallas_call(
        _final_kernel,
        grid=(1,),
        in_specs=[
            pl.BlockSpec((N, H), lambda b: (0, 0)),
            pl.BlockSpec((1, N), lambda b: (0, 0)),
            pl.BlockSpec((1, H), lambda b: (0, 0)),
            pl.BlockSpec((H, H), lambda b: (0, 0)),
            pl.BlockSpec((1, H), lambda b: (0, 0)),
            pl.BlockSpec((H, 4 * (H // 2)), lambda b: (0, 0)),
            pl.BlockSpec((1, 4 * (H // 2)), lambda b: (0, 0)),
            pl.BlockSpec((1, 4 * (H // 2)), lambda b: (0, 0)),
            pl.BlockSpec((1, 4), lambda b: (0, 0)),
        ],
        out_specs=pl.BlockSpec((1, 4), lambda b: (0, 0)),
        out_shape=jax.ShapeDtypeStruct((1, 4), f32),
    )(z, wmask.reshape(1, N), bn1r, Wn2, bn2r, Wh1r, bh1r, Wh2r, bh2r)

    return out.reshape(4)
